# baseline (speedup 1.0000x reference)
import jax
import jax.numpy as jnp
from jax.experimental import pallas as pl
from jax.experimental.pallas import tpu as pltpu


def _copy_body(w_hbm, o_hbm, w_vmem, in_sem, out_sem, *, B, n_chunks, CT):
    def start_in(k, slot):
        pltpu.make_async_copy(
            w_hbm.at[pl.ds(k * CT, CT), :], w_vmem.at[slot], in_sem.at[slot]
        ).start()

    start_in(0, 0)

    def loop(k, _):
        slot = jax.lax.rem(k, 2)
        @pl.when(k + 1 < n_chunks)
        def _():
            start_in(k + 1, 1 - slot)

        pltpu.make_async_copy(
            w_hbm.at[pl.ds(k * CT, CT), :], w_vmem.at[slot], in_sem.at[slot]
        ).wait()
        for b in range(B):
            pltpu.make_async_copy(
                w_vmem.at[slot], o_hbm.at[b, pl.ds(k * CT, CT), :], out_sem.at[b]
            ).start()
        for b in range(B):
            pltpu.make_async_copy(
                w_vmem.at[slot], o_hbm.at[b, pl.ds(k * CT, CT), :], out_sem.at[b]
            ).wait()
        return 0

    jax.lax.fori_loop(0, n_chunks, loop, 0)


def kernel(x, W):
    B = x.shape[0]
    T, H = W.shape
    CT = 512
    n_chunks = T // CT
    import functools

    body = functools.partial(_copy_body, B=B, n_chunks=n_chunks, CT=CT)
    return pl.pallas_call(
        body,
        in_specs=[pl.BlockSpec(memory_space=pl.ANY)],
        out_specs=pl.BlockSpec(memory_space=pl.ANY),
        out_shape=jax.ShapeDtypeStruct((B, T, H), W.dtype),
        scratch_shapes=[
            pltpu.VMEM((2, CT, H), W.dtype),
            pltpu.SemaphoreType.DMA((2,)),
            pltpu.SemaphoreType.DMA((B,)),
        ],
    )(W)


# speedup vs baseline: 1.1465x; 1.1465x over previous
import functools

import jax
import jax.numpy as jnp
from jax.experimental import pallas as pl
from jax.experimental.pallas import tpu as pltpu


def _copy_body(w_hbm, o_hbm, w_vmem, in_sem, out_sem, *, B, K, CT):
    ins = [
        pltpu.make_async_copy(
            w_hbm.at[pl.ds(k * CT, CT), :],
            w_vmem.at[pl.ds(k * CT, CT), :],
            in_sem.at[k],
        )
        for k in range(K)
    ]
    for c in ins:
        c.start()
    outs = []
    for k in range(K):
        ins[k].wait()
        for b in range(B):
            c = pltpu.make_async_copy(
                w_vmem.at[pl.ds(k * CT, CT), :],
                o_hbm.at[b, pl.ds(k * CT, CT), :],
                out_sem.at[b],
            )
            c.start()
            outs.append(c)
    for c in outs:
        c.wait()


def kernel(x, W):
    B = x.shape[0]
    T, H = W.shape
    K = 4
    CT = T // K
    body = functools.partial(_copy_body, B=B, K=K, CT=CT)
    return pl.pallas_call(
        body,
        in_specs=[pl.BlockSpec(memory_space=pl.ANY)],
        out_specs=pl.BlockSpec(memory_space=pl.ANY),
        out_shape=jax.ShapeDtypeStruct((B, T, H), W.dtype),
        scratch_shapes=[
            pltpu.VMEM((T, H), W.dtype),
            pltpu.SemaphoreType.DMA((K,)),
            pltpu.SemaphoreType.DMA((B,)),
        ],
    )(W)
